# Initial kernel scaffold; baseline (speedup 1.0000x reference)
import jax
import jax.numpy as jnp
from jax.experimental import pallas as pl


def kernel(input, weight, bias):
    raise NotImplementedError("write your pallas kernel here")


# speedup vs baseline: 24.1786x; 24.1786x over previous
import jax
import jax.numpy as jnp
from jax.experimental import pallas as pl
from jax.experimental.pallas import tpu as pltpu

_TILE = 256
_BM = 512


def _select_body(w_ref, t_ref, bits_ref, k_rank):
    bits_ref[...] = jax.lax.bitcast_convert_type(
        w_ref[...], jnp.int32) & jnp.int32(0x7FFFFFFF)

    def step(i, prefix):
        trial = prefix + jax.lax.shift_left(jnp.int32(1), jnp.int32(30) - i)
        c = jnp.sum((bits_ref[...] < trial).astype(jnp.int32))
        return jnp.where(c <= k_rank, trial, prefix)

    prefix = jax.lax.fori_loop(0, 31, step, jnp.int32(0))
    t_ref[0, 0] = jax.lax.bitcast_convert_type(prefix, jnp.float32)


def _maskcast_body(t_ref, w_ref, out_ref):
    w = w_ref[...]
    t = t_ref[0, 0]
    wm = jnp.where(jnp.abs(w) > t, w, 0.0)
    out_ref[...] = wm.T.astype(jnp.bfloat16)


def _matmul_body(x_ref, wt_ref, b_ref, out_ref):
    xb = x_ref[...].astype(jnp.bfloat16)
    acc = jnp.dot(xb, wt_ref[...], preferred_element_type=jnp.float32)
    out_ref[...] = acc + b_ref[...]


def kernel(input, weight, bias):
    n_out, n_in = weight.shape
    x2d = input.reshape(-1, n_in)
    m = x2d.shape[0]
    k_rank = (n_out * n_in) // 2 - 1

    threshold = pl.pallas_call(
        lambda w_ref, t_ref, bits_ref: _select_body(w_ref, t_ref, bits_ref,
                                                    k_rank),
        out_shape=jax.ShapeDtypeStruct((1, 1), jnp.float32),
        in_specs=[pl.BlockSpec((n_out, n_in), lambda: (0, 0))],
        out_specs=pl.BlockSpec(memory_space=pltpu.SMEM),
        scratch_shapes=[pltpu.VMEM((n_out, n_in), jnp.int32)],
    )(weight)

    wt = pl.pallas_call(
        _maskcast_body,
        grid=(n_out // _TILE, n_in // _TILE),
        in_specs=[
            pl.BlockSpec(memory_space=pltpu.SMEM),
            pl.BlockSpec((_TILE, _TILE), lambda i, j: (i, j)),
        ],
        out_specs=pl.BlockSpec((_TILE, _TILE), lambda i, j: (j, i)),
        out_shape=jax.ShapeDtypeStruct((n_in, n_out), jnp.bfloat16),
    )(threshold, weight)

    out = pl.pallas_call(
        _matmul_body,
        grid=(m // _BM,),
        in_specs=[
            pl.BlockSpec((_BM, n_in), lambda i: (i, 0)),
            pl.BlockSpec((n_in, n_out), lambda i: (0, 0)),
            pl.BlockSpec((1, n_out), lambda i: (0, 0)),
        ],
        out_specs=pl.BlockSpec((_BM, n_out), lambda i: (i, 0)),
        out_shape=jax.ShapeDtypeStruct((m, n_out), jnp.float32),
    )(x2d, wt, bias.reshape(1, n_out))

    return out.reshape(*input.shape[:-1], n_out)


# speedup vs baseline: 30.3968x; 1.2572x over previous
import jax
import jax.numpy as jnp
from jax.experimental import pallas as pl
from jax.experimental.pallas import tpu as pltpu

_BM = 512


def _count_below(s16_ref, t16):
    rows, _ = s16_ref.shape
    chunk = 512
    total = jnp.int32(0)
    for r in range(rows // chunk):
        cs = jnp.sum(
            (s16_ref[pl.ds(r * chunk, chunk), :] < t16).astype(jnp.int16),
            axis=0)
        total = total + jnp.sum(cs.astype(jnp.int32))
    return total


def _fused_body(x_ref, w_ref, b_ref, out_ref, s16_ref, wt_ref, k_rank):
    i = pl.program_id(0)

    n_out, n_in = w_ref.shape
    tile = 256

    @pl.when(i == 0)
    def _select_and_mask():
        for r in range(n_out // tile):
            bits = jax.lax.bitcast_convert_type(
                w_ref[pl.ds(r * tile, tile), :],
                jnp.int32) & jnp.int32(0x7FFFFFFF)
            s16_ref[pl.ds(r * tile, tile), :] = jax.lax.shift_right_logical(
                bits, jnp.int32(16)).astype(jnp.int16)

        def step_a(j, prefix):
            trial = prefix + jax.lax.shift_left(jnp.int32(1),
                                                jnp.int32(14) - j)
            c = _count_below(s16_ref, trial.astype(jnp.int16))
            return jnp.where(c <= k_rank, trial, prefix)

        p_hi = jax.lax.fori_loop(0, 15, step_a, jnp.int32(0))
        cb = _count_below(s16_ref, p_hi.astype(jnp.int16))
        k2 = k_rank - cb

        for r in range(n_out // tile):
            rs = pl.ds(r * tile, tile)
            rbits = jax.lax.bitcast_convert_type(
                w_ref[rs, :], jnp.int32) & jnp.int32(0x7FFFFFFF)
            lo16 = ((rbits & jnp.int32(0xFFFF)) - jnp.int32(32768)).astype(
                jnp.int16)
            s16_ref[rs, :] = jnp.where(
                s16_ref[rs, :] == p_hi.astype(jnp.int16), lo16,
                jnp.int16(32767))

        def step_b(j, prefix):
            trial = prefix + jax.lax.shift_left(jnp.int32(1),
                                                jnp.int32(15) - j)
            t16 = (trial - jnp.int32(32768)).astype(jnp.int16)
            c = _count_below(s16_ref, t16)
            return jnp.where(c <= k2, trial, prefix)

        p_lo = jax.lax.fori_loop(0, 16, step_b, jnp.int32(0))

        tbits = jax.lax.shift_left(p_hi, jnp.int32(16)) | p_lo
        t = jax.lax.bitcast_convert_type(tbits, jnp.float32)
        for ti in range(n_out // tile):
            for tj in range(n_in // tile):
                wtile = w_ref[pl.ds(ti * tile, tile), pl.ds(tj * tile, tile)]
                wm = jnp.where(jnp.abs(wtile) > t, wtile, 0.0)
                wt_ref[pl.ds(tj * tile, tile), pl.ds(ti * tile, tile)] = (
                    wm.T.astype(jnp.bfloat16))

    @pl.when(i > 0)
    def _gemm():
        xb = x_ref[...].astype(jnp.bfloat16)
        acc = jnp.dot(xb, wt_ref[...], preferred_element_type=jnp.float32)
        out_ref[...] = acc + b_ref[...]


def kernel(input, weight, bias):
    n_out, n_in = weight.shape
    x2d = input.reshape(-1, n_in)
    m = x2d.shape[0]
    k_rank = (n_out * n_in) // 2 - 1

    out = pl.pallas_call(
        lambda x_ref, w_ref, b_ref, out_ref, s16_ref, wt_ref: _fused_body(
            x_ref, w_ref, b_ref, out_ref, s16_ref, wt_ref, k_rank),
        grid=(1 + m // _BM,),
        in_specs=[
            pl.BlockSpec((_BM, n_in), lambda i: (jnp.maximum(i - 1, 0), 0)),
            pl.BlockSpec((n_out, n_in), lambda i: (0, 0)),
            pl.BlockSpec((1, n_out), lambda i: (0, 0)),
        ],
        out_specs=pl.BlockSpec((_BM, n_out),
                               lambda i: (jnp.maximum(i - 1, 0), 0)),
        out_shape=jax.ShapeDtypeStruct((m, n_out), jnp.float32),
        scratch_shapes=[
            pltpu.VMEM((n_out, n_in), jnp.int16),
            pltpu.VMEM((n_in, n_out), jnp.bfloat16),
        ],
    )(x2d, weight, bias.reshape(1, n_out))

    return out.reshape(*input.shape[:-1], n_out)
